# Initial kernel scaffold; baseline (speedup 1.0000x reference)
import jax
import jax.numpy as jnp
from jax.experimental import pallas as pl


def kernel(x, table, W, b):
    raise NotImplementedError("write your pallas kernel here")


# speedup vs baseline: 2.0157x; 2.0157x over previous
import functools

import jax
import jax.numpy as jnp
from jax import lax
from jax.experimental import pallas as pl
from jax.experimental.pallas import tpu as pltpu
from jax.experimental.pallas import tpu_sc as plsc

PAD = 100000
D = 128
L = 200
LP = 208
HALF = LP // 2
BATCH = 4096
NOUT = 100
NCORES = 2
NSUB = 16
NW = NCORES * NSUB
RPW = BATCH // NW
LANES = 16
NCH = D // LANES


def _sc_rowsum(xp, table):
    mesh = plsc.VectorSubcoreMesh(
        core_axis_name="c", subcore_axis_name="s",
        num_cores=NCORES, num_subcores=NSUB)

    @functools.partial(
        pl.kernel,
        out_type=jax.ShapeDtypeStruct((BATCH, D), jnp.float32),
        mesh=mesh,
        scratch_types=[
            pltpu.VMEM((2, HALF), jnp.int32),
            pltpu.VMEM((2, HALF), jnp.int32),
            pltpu.VMEM((LP, D), jnp.float32),
            pltpu.VMEM((LP, D), jnp.float32),
            pltpu.VMEM((RPW, D), jnp.float32),
            pltpu.SemaphoreType.DMA,
            pltpu.SemaphoreType.DMA,
        ],
    )
    def k(x_hbm, table_hbm, out_hbm, idx0, idx1, rows0, rows1, out_v, sem0, sem1):
        wid = lax.axis_index("s") * NCORES + lax.axis_index("c")
        base = wid * RPW

        def gather_copies(idx_v, rows_v, sem):
            return [
                pltpu.make_async_copy(
                    table_hbm.at[idx_v.at[j]],
                    rows_v.at[pl.ds(j * HALF, HALF)],
                    sem)
                for j in range(2)
            ]

        def start(r, idx_v, rows_v, sem):
            pltpu.sync_copy(x_hbm.at[r], idx_v)
            for cp in gather_copies(idx_v, rows_v, sem):
                cp.start()

        def wait(idx_v, rows_v, sem):
            for cp in gather_copies(idx_v, rows_v, sem):
                cp.wait()

        def accum(rows_v, i):
            def body(t, accs):
                return tuple(
                    accs[c] + rows_v[t, pl.ds(c * LANES, LANES)]
                    for c in range(NCH))
            accs = tuple(jnp.zeros((LANES,), jnp.float32) for _ in range(NCH))
            accs = lax.fori_loop(0, LP, body, accs)
            for c in range(NCH):
                out_v[i, pl.ds(c * LANES, LANES)] = accs[c]

        start(base, idx0, rows0, sem0)

        def loop_body(j, carry):
            a = 2 * j
            start(base + a + 1, idx1, rows1, sem1)
            wait(idx0, rows0, sem0)
            accum(rows0, a)

            @pl.when(j < RPW // 2 - 1)
            def _():
                start(base + a + 2, idx0, rows0, sem0)

            wait(idx1, rows1, sem1)
            accum(rows1, a + 1)
            return carry

        lax.fori_loop(0, RPW // 2, loop_body, 0)
        pltpu.sync_copy(out_v, out_hbm.at[pl.ds(base, RPW)])

    return k(xp, table)


def _tc_head(x, summed, W, b):
    blk = 512
    grid = BATCH // blk

    def body(x_ref, s_ref, w_ref, b_ref, o_ref):
        cnt = jnp.sum((x_ref[...] != PAD).astype(jnp.float32),
                      axis=1, keepdims=True)
        mean = s_ref[...] / jnp.maximum(cnt, 1e-6)
        o_ref[...] = jnp.dot(mean, w_ref[...],
                             preferred_element_type=jnp.float32) + b_ref[...]

    return pl.pallas_call(
        body,
        grid=(grid,),
        in_specs=[
            pl.BlockSpec((blk, L), lambda i: (i, 0)),
            pl.BlockSpec((blk, D), lambda i: (i, 0)),
            pl.BlockSpec((D, NOUT), lambda i: (0, 0)),
            pl.BlockSpec((1, NOUT), lambda i: (0, 0)),
        ],
        out_specs=pl.BlockSpec((blk, NOUT), lambda i: (i, 0)),
        out_shape=jax.ShapeDtypeStruct((BATCH, NOUT), jnp.float32),
    )(x, summed, W, b.reshape(1, NOUT))


def kernel(x, table, W, b):
    xp = jnp.pad(x, ((0, 0), (0, LP - L)), constant_values=PAD)
    xp = xp.reshape(BATCH, 2, HALF)
    summed = _sc_rowsum(xp, table)
    return _tc_head(x, summed, W, b)
